# baseline (speedup 1.0000x reference)
import jax
import jax.numpy as jnp
from jax.experimental import pallas as pl

_N = 2048
_D = 2048
_E = 64
_K = 8
_NG = 8
_TG = 4
_GS = _E // _NG
_F = 128
_SCALE = 2.5
_NEG = -1e30

_EBA = 4
_TT = 1024
_NA = _E // _EBA
_KCE = 8
_NB = _E // _KCE
_HW = (_NA + 2) * _EBA * _F


def _gate_kernel(x_ref, gw_ref, eb_ref, w_ref):
    x = x_ref[:]
    logits = jnp.dot(x, gw_ref[:], preferred_element_type=jnp.float32)
    scores = jax.nn.sigmoid(logits)
    sc = scores + eb_ref[:]

    lane = jax.lax.broadcasted_iota(jnp.int32, (1, _E), 1)
    lane_g = lane // _GS

    gcols = []
    for g in range(_NG):
        gm = lane_g == g
        m1 = jnp.max(jnp.where(gm, sc, _NEG), axis=1, keepdims=True)
        m2 = jnp.max(jnp.where(gm & (sc < m1), sc, _NEG), axis=1, keepdims=True)
        gcols.append(m1 + m2)
    gs = jnp.concatenate(gcols, axis=1)

    glane = jax.lax.broadcasted_iota(jnp.int32, (1, _NG), 1)
    sel = jnp.zeros(gs.shape, dtype=jnp.bool_)
    cur = gs
    for _ in range(_TG):
        m = jnp.max(cur, axis=1, keepdims=True)
        hit = cur == m
        hp = jnp.min(jnp.where(hit, glane, _NG), axis=1, keepdims=True)
        hit1 = glane == hp
        sel = sel | hit1
        cur = jnp.where(hit1, _NEG, cur)

    emask = jnp.zeros(sc.shape, dtype=jnp.bool_)
    for g in range(_NG):
        emask = emask | ((lane_g == g) & sel[:, g:g + 1])

    masked = jnp.where(emask, sc, _NEG)
    chosen = jnp.zeros(sc.shape, dtype=jnp.bool_)
    cur = masked
    for _ in range(_K):
        m = jnp.max(cur, axis=1, keepdims=True)
        hit = cur == m
        hp = jnp.min(jnp.where(hit, lane, _E), axis=1, keepdims=True)
        hit1 = lane == hp
        chosen = chosen | hit1
        cur = jnp.where(hit1, _NEG, cur)

    wsel = jnp.where(chosen, scores, 0.0)
    wsum = jnp.sum(wsel, axis=1, keepdims=True) + 1e-20
    w = wsel / wsum * _SCALE
    zpad = jnp.zeros((w.shape[0], 128 - _E), dtype=jnp.float32)
    w_ref[:] = jnp.concatenate([w, zpad], axis=1)


def _up_kernel(w_all_ref, x_ref, w1_ref, w3_ref, sw13_ref, h_ref):
    eb = pl.program_id(1)
    x = x_ref[:]

    @pl.when(eb < _NA)
    def _routed():
        w13 = jnp.concatenate(
            [w1_ref[j].astype(jnp.bfloat16) for j in range(_EBA)]
            + [w3_ref[j].astype(jnp.bfloat16) for j in range(_EBA)], axis=1)
        h13 = jnp.dot(x, w13, preferred_element_type=jnp.float32)

        lane = jax.lax.broadcasted_iota(jnp.int32, (1, 128), 1)
        w_all = w_all_ref[:]
        hs = []
        for j in range(_EBA):
            h1 = h13[:, j * _F:(j + 1) * _F]
            h3 = h13[:, (_EBA + j) * _F:(_EBA + j + 1) * _F]
            we = jnp.sum(jnp.where(lane == eb * _EBA + j, w_all, 0.0),
                         axis=1, keepdims=True)
            hs.append(jax.nn.silu(h1) * h3 * we)
        h_ref[:] = jnp.concatenate(hs, axis=1).astype(jnp.bfloat16)

    @pl.when(eb == _NA)
    def _shared():
        h13 = jnp.dot(x, sw13_ref[:].astype(jnp.bfloat16),
                      preferred_element_type=jnp.float32)
        hsh = jax.nn.silu(h13[:, :_F]) * h13[:, _F:]
        pad = jnp.zeros((_TT, (_EBA - 1) * _F), dtype=jnp.float32)
        h_ref[:] = jnp.concatenate([hsh, pad], axis=1).astype(jnp.bfloat16)

    @pl.when(eb == _NA + 1)
    def _zfill():
        h_ref[:] = jnp.zeros((_TT, _EBA * _F), dtype=jnp.bfloat16)


def _down_kernel(h_ref, w2_ref, sw2_ref, out_ref):
    kc = pl.program_id(1)

    @pl.when(kc < _NB)
    def _routed():
        w2m = w2_ref[:].reshape(_KCE * _F, _D).astype(jnp.bfloat16)
        o = jnp.dot(h_ref[:], w2m, preferred_element_type=jnp.float32)

        @pl.when(kc == 0)
        def _():
            out_ref[:] = o

        @pl.when(kc > 0)
        def _():
            out_ref[:] = out_ref[:] + o

    @pl.when(kc == _NB)
    def _shared():
        o = jnp.dot(h_ref[:, :_F], sw2_ref[:].astype(jnp.bfloat16),
                    preferred_element_type=jnp.float32)
        out_ref[:] = out_ref[:] + o


def kernel(hidden_states, gate_w, e_bias, w1, w3, w2, sw1, sw3, sw2):
    x = hidden_states.reshape(_N, _D)

    gt = 256
    w_all = pl.pallas_call(
        _gate_kernel,
        grid=(_N // gt,),
        in_specs=[
            pl.BlockSpec((gt, _D), lambda i: (i, 0)),
            pl.BlockSpec((_D, _E), lambda i: (0, 0)),
            pl.BlockSpec((1, _E), lambda i: (0, 0)),
        ],
        out_specs=pl.BlockSpec((gt, 128), lambda i: (i, 0)),
        out_shape=jax.ShapeDtypeStruct((_N, 128), jnp.float32),
    )(x, gate_w.T, e_bias.reshape(1, _E))

    xb = x.astype(jnp.bfloat16)
    nt = _N // _TT
    ca = _NA - 1

    h_all = pl.pallas_call(
        _up_kernel,
        grid=(nt, _NA + 2),
        in_specs=[
            pl.BlockSpec((_TT, 128), lambda t, eb: (t, 0)),
            pl.BlockSpec((_TT, _D), lambda t, eb: (t, 0)),
            pl.BlockSpec((_EBA, _D, _F),
                         lambda t, eb: (jnp.minimum(eb, ca), 0, 0)),
            pl.BlockSpec((_EBA, _D, _F),
                         lambda t, eb: (jnp.minimum(eb, ca), 0, 0)),
            pl.BlockSpec((_D, 2 * _F), lambda t, eb: (0, 0)),
        ],
        out_specs=pl.BlockSpec((_TT, _EBA * _F), lambda t, eb: (t, eb)),
        out_shape=jax.ShapeDtypeStruct((_N, _HW), jnp.bfloat16),
    )(w_all, xb, w1, w3, jnp.concatenate([sw1, sw3], axis=1))

    cb = _NB - 1
    out = pl.pallas_call(
        _down_kernel,
        grid=(nt, _NB + 1),
        in_specs=[
            pl.BlockSpec((_TT, _KCE * _F), lambda t, kc: (t, kc)),
            pl.BlockSpec((_KCE, _F, _D),
                         lambda t, kc: (jnp.minimum(kc, cb), 0, 0)),
            pl.BlockSpec((_F, _D), lambda t, kc: (0, 0)),
        ],
        out_specs=pl.BlockSpec((_TT, _D), lambda t, kc: (t, 0)),
        out_shape=jax.ShapeDtypeStruct((_N, _D), jnp.float32),
    )(h_all, w2, sw2)

    return out.reshape(hidden_states.shape)


# speedup vs baseline: 1.0335x; 1.0335x over previous
import jax
import jax.numpy as jnp
from jax.experimental import pallas as pl

_N = 2048
_D = 2048
_E = 64
_K = 8
_NG = 8
_TG = 4
_GS = _E // _NG
_F = 128
_SCALE = 2.5
_NEG = -1e30

_EBA = 4
_TT = 1024
_NA = _E // _EBA
_KCE = 8
_NB = _E // _KCE
_HW = (_NA + 2) * _EBA * _F


def _gate_kernel(x_ref, gw_ref, eb_ref, w_ref):
    x = x_ref[:]
    logits = jnp.dot(x, gw_ref[:], preferred_element_type=jnp.float32)
    scores = jax.nn.sigmoid(logits)
    sc = scores + eb_ref[:]

    lane = jax.lax.broadcasted_iota(jnp.int32, (1, _E), 1)
    lane_g = lane // _GS

    gcols = []
    for g in range(_NG):
        gm = lane_g == g
        m1 = jnp.max(jnp.where(gm, sc, _NEG), axis=1, keepdims=True)
        m2 = jnp.max(jnp.where(gm & (sc < m1), sc, _NEG), axis=1, keepdims=True)
        gcols.append(m1 + m2)
    gs = jnp.concatenate(gcols, axis=1)

    glane = jax.lax.broadcasted_iota(jnp.int32, (1, _NG), 1)
    sel = jnp.zeros(gs.shape, dtype=jnp.bool_)
    cur = gs
    for _ in range(_TG):
        m = jnp.max(cur, axis=1, keepdims=True)
        hit = cur == m
        hp = jnp.min(jnp.where(hit, glane, _NG), axis=1, keepdims=True)
        hit1 = glane == hp
        sel = sel | hit1
        cur = jnp.where(hit1, _NEG, cur)

    emask = jnp.zeros(sc.shape, dtype=jnp.bool_)
    for g in range(_NG):
        emask = emask | ((lane_g == g) & sel[:, g:g + 1])

    masked = jnp.where(emask, sc, _NEG)
    chosen = jnp.zeros(sc.shape, dtype=jnp.bool_)
    cur = masked
    for _ in range(_K):
        m = jnp.max(cur, axis=1, keepdims=True)
        hit = cur == m
        hp = jnp.min(jnp.where(hit, lane, _E), axis=1, keepdims=True)
        hit1 = lane == hp
        chosen = chosen | hit1
        cur = jnp.where(hit1, _NEG, cur)

    wsel = jnp.where(chosen, scores, 0.0)
    wsum = jnp.sum(wsel, axis=1, keepdims=True) + 1e-20
    w = wsel / wsum * _SCALE
    zpad = jnp.zeros((w.shape[0], 128 - _E), dtype=jnp.float32)
    w_ref[:] = jnp.concatenate([w, zpad], axis=1)


def _up_kernel(w_all_ref, x_ref, w1_ref, w3_ref, sw13_ref, h_ref):
    eb = pl.program_id(1)
    x = x_ref[:]

    @pl.when(eb < _NA)
    def _routed():
        w13 = jnp.concatenate(
            [w1_ref[j].astype(jnp.bfloat16) for j in range(_EBA)]
            + [w3_ref[j].astype(jnp.bfloat16) for j in range(_EBA)], axis=1)
        h13 = jnp.dot(x, w13, preferred_element_type=jnp.float32)

        lane = jax.lax.broadcasted_iota(jnp.int32, (1, 128), 1)
        w_all = w_all_ref[:]
        hs = []
        for j in range(_EBA):
            h1 = h13[:, j * _F:(j + 1) * _F]
            h3 = h13[:, (_EBA + j) * _F:(_EBA + j + 1) * _F]
            we = jnp.sum(jnp.where(lane == eb * _EBA + j, w_all, 0.0),
                         axis=1, keepdims=True)
            hs.append(jax.nn.silu(h1) * h3 * we)
        h_ref[:] = jnp.concatenate(hs, axis=1).astype(jnp.bfloat16)

    @pl.when(eb == _NA)
    def _shared():
        h13 = jnp.dot(x, sw13_ref[:].astype(jnp.bfloat16),
                      preferred_element_type=jnp.float32)
        hsh = jax.nn.silu(h13[:, :_F]) * h13[:, _F:]
        pad = jnp.zeros((_TT, (_EBA - 1) * _F), dtype=jnp.float32)
        h_ref[:] = jnp.concatenate([hsh, pad], axis=1).astype(jnp.bfloat16)

    @pl.when(eb == _NA + 1)
    def _zfill():
        h_ref[:] = jnp.zeros((_TT, _EBA * _F), dtype=jnp.bfloat16)


def _down_kernel(h_ref, w2_ref, sw2_ref, out_ref):
    kc = pl.program_id(1)

    @pl.when(kc < _NB)
    def _routed():
        w2m = w2_ref[:].reshape(_KCE * _F, _D).astype(jnp.bfloat16)
        o = jnp.dot(h_ref[:], w2m, preferred_element_type=jnp.float32)

        @pl.when(kc == 0)
        def _():
            out_ref[:] = o

        @pl.when(kc > 0)
        def _():
            out_ref[:] = out_ref[:] + o

    @pl.when(kc == _NB)
    def _shared():
        o = jnp.dot(h_ref[:, :_F], sw2_ref[:].astype(jnp.bfloat16),
                    preferred_element_type=jnp.float32)
        out_ref[:] = out_ref[:] + o


def kernel(hidden_states, gate_w, e_bias, w1, w3, w2, sw1, sw3, sw2):
    x = hidden_states.reshape(_N, _D)

    w_all = pl.pallas_call(
        _gate_kernel,
        out_shape=jax.ShapeDtypeStruct((_N, 128), jnp.float32),
    )(x, gate_w.T, e_bias.reshape(1, _E))

    xb = x.astype(jnp.bfloat16)
    nt = _N // _TT
    ca = _NA - 1

    h_all = pl.pallas_call(
        _up_kernel,
        grid=(nt, _NA + 2),
        in_specs=[
            pl.BlockSpec((_TT, 128), lambda t, eb: (t, 0)),
            pl.BlockSpec((_TT, _D), lambda t, eb: (t, 0)),
            pl.BlockSpec((_EBA, _D, _F),
                         lambda t, eb: (jnp.minimum(eb, ca), 0, 0)),
            pl.BlockSpec((_EBA, _D, _F),
                         lambda t, eb: (jnp.minimum(eb, ca), 0, 0)),
            pl.BlockSpec((_D, 2 * _F), lambda t, eb: (0, 0)),
        ],
        out_specs=pl.BlockSpec((_TT, _EBA * _F), lambda t, eb: (t, eb)),
        out_shape=jax.ShapeDtypeStruct((_N, _HW), jnp.bfloat16),
    )(w_all, xb, w1, w3, jnp.concatenate([sw1, sw3], axis=1))

    cb = _NB - 1
    out = pl.pallas_call(
        _down_kernel,
        grid=(nt, _NB + 1),
        in_specs=[
            pl.BlockSpec((_TT, _KCE * _F), lambda t, kc: (t, kc)),
            pl.BlockSpec((_KCE, _F, _D),
                         lambda t, kc: (jnp.minimum(kc, cb), 0, 0)),
            pl.BlockSpec((_F, _D), lambda t, kc: (0, 0)),
        ],
        out_specs=pl.BlockSpec((_TT, _D), lambda t, kc: (t, 0)),
        out_shape=jax.ShapeDtypeStruct((_N, _D), jnp.float32),
    )(h_all, w2, sw2)

    return out.reshape(hidden_states.shape)
